# baseline (device time: 12007 ns/iter reference)
import jax
import jax.numpy as jnp
from jax import lax
from jax.experimental import pallas as pl
from jax.experimental.pallas import tpu as pltpu

BM = 256


def kernel(x, dy, gamma):
    m, d = x.shape
    n_chunks = m // BM

    def body(x_ref, dy_ref, gamma_ref, out_ref, acc_ref, recv_ref,
             send_sem, recv_sem):
        i = pl.program_id(0)
        my_x = lax.axis_index("x")
        my_y = lax.axis_index("y")
        my_z = lax.axis_index("z")
        barrier_sem = pltpu.get_barrier_semaphore()

        xv = x_ref[:, :]
        dyv = dy_ref[:, :]
        mu = jnp.mean(xv, axis=1, keepdims=True)
        xc = xv - mu
        var = jnp.mean(xc * xc, axis=1, keepdims=True)
        rstd = lax.rsqrt(var + 1e-5)
        q = dyv * xc * rstd

        ones_row = jnp.ones((1, BM), jnp.float32)
        dg = jax.lax.dot_general(
            ones_row, q, (((1,), (0,)), ((), ())),
            preferred_element_type=jnp.float32,
        )
        db = jax.lax.dot_general(
            ones_row, dyv, (((1,), (0,)), ((), ())),
            preferred_element_type=jnp.float32,
        )

        @pl.when(i == 0)
        def _():
            acc_ref[0:1, :] = dg
            acc_ref[1:2, :] = db

        @pl.when(i > 0)
        def _():
            acc_ref[0:1, :] = acc_ref[0:1, :] + dg
            acc_ref[1:2, :] = acc_ref[1:2, :] + db

        @pl.when(i == n_chunks - 1)
        def _():
            pl.semaphore_signal(
                barrier_sem, inc=1,
                device_id=(1 - my_x, my_y, my_z),
                device_id_type=pl.DeviceIdType.MESH,
            )
            pl.semaphore_wait(barrier_sem, 1)

            rdma = pltpu.make_async_remote_copy(
                src_ref=acc_ref,
                dst_ref=recv_ref,
                send_sem=send_sem,
                recv_sem=recv_sem,
                device_id=(1 - my_x, my_y, my_z),
                device_id_type=pl.DeviceIdType.MESH,
            )
            rdma.start()
            rdma.wait()

            out_ref[:, :] = acc_ref[:, :] + recv_ref[:, :]

    return pl.pallas_call(
        body,
        grid=(n_chunks,),
        out_shape=jax.ShapeDtypeStruct((2, d), jnp.float32),
        in_specs=[
            pl.BlockSpec((BM, d), lambda i: (i, 0)),
            pl.BlockSpec((BM, d), lambda i: (i, 0)),
            pl.BlockSpec(memory_space=pltpu.MemorySpace.HBM),
        ],
        out_specs=pl.BlockSpec((2, d), lambda i: (0, 0)),
        scratch_shapes=[
            pltpu.VMEM((2, d), jnp.float32),
            pltpu.VMEM((2, d), jnp.float32),
            pltpu.SemaphoreType.DMA,
            pltpu.SemaphoreType.DMA,
        ],
        compiler_params=pltpu.CompilerParams(
            collective_id=0,
            dimension_semantics=("arbitrary",),
        ),
    )(x, dy, gamma)
